# baseline (speedup 1.0000x reference)
import jax
import jax.numpy as jnp
from jax import lax
from jax.experimental import pallas as pl
from jax.experimental.pallas import tpu as pltpu
from jax.experimental.pallas import tpu_sc as plsc

_MAX_LENGTH = 2048
_BOS = 49406
_EOS = 49407
_PAD = 49407
_B = 16
_TOTAL = 16384

_HALF = _MAX_LENGTH // 2
_WIN = _HALF + 16
_WALLOC = 3104


def _body(flat_hbm, cu_hbm, ids_hbm, mask_hbm,
          win_v, cu_v, outi_v, outm_v, sem_w, sem_m):
    row = lax.axis_index("s")
    half = lax.axis_index("c")
    p0 = half * _HALF

    pltpu.sync_copy(cu_hbm, cu_v.at[pl.ds(0, _B + 1)])
    cu_vec = cu_v[pl.ds(row, 16)]
    start = cu_vec[0]
    end = cu_vec[1]
    seg = jnp.minimum(end - start, _MAX_LENGTH - 2)

    aa = start + p0 - 1
    aa0 = jnp.maximum(aa, 0)
    r0 = jnp.minimum(aa0 // 16, (_TOTAL - _WIN) // 16) * 16
    off = aa - r0 + 16
    wcopy = pltpu.async_copy(
        flat_hbm.at[pl.ds(r0, _WIN)], win_v.at[pl.ds(16, _WIN)], sem_w)

    lane = lax.iota(jnp.int32, 16)
    one = jnp.int32(1)
    zero = jnp.int32(0)

    @plsc.parallel_loop(0, _HALF, step=16, unroll=8)
    def _mask_loop(i):
        p = p0 + i + lane
        outm_v[pl.ds(i, 16)] = jnp.maximum(
            jnp.minimum(seg + 2 - p, one), zero)

    mcopy = pltpu.async_copy(
        outm_v, mask_hbm.at[row, pl.ds(p0, _HALF)], sem_m)
    wcopy.wait()

    def frame(p, g):
        inb = jnp.maximum(
            jnp.minimum(jnp.minimum(p, seg + 1 - p), one), zero)
        a = jnp.minimum(p, one)
        return inb * (g - _PAD) + (a + (_PAD - 1))

    @plsc.parallel_loop(0, _HALF // 2, step=16, unroll=8)
    def _ids_loop_a(i):
        p = p0 + i + lane
        g = win_v[pl.ds(off + i, 16)]
        outi_v[pl.ds(i, 16)] = frame(p, g)

    icopy_a = pltpu.async_copy(
        outi_v.at[pl.ds(0, _HALF // 2)],
        ids_hbm.at[row, pl.ds(p0, _HALF // 2)], sem_w)

    @plsc.parallel_loop(_HALF // 2, _HALF, step=16, unroll=8)
    def _ids_loop_b(i):
        p = p0 + i + lane
        g = win_v[pl.ds(off + i, 16)]
        outi_v[pl.ds(i, 16)] = frame(p, g)

    icopy_b = pltpu.async_copy(
        outi_v.at[pl.ds(_HALF // 2, _HALF // 2)],
        ids_hbm.at[row, pl.ds(p0 + _HALF // 2, _HALF // 2)], sem_w)
    icopy_a.wait()
    icopy_b.wait()
    mcopy.wait()


@jax.jit
def kernel(flat_tokens, cu_seqlens):
    mesh = plsc.VectorSubcoreMesh(
        core_axis_name="c", subcore_axis_name="s", num_cores=2, num_subcores=16)
    run = pl.kernel(
        _body,
        out_type=(
            jax.ShapeDtypeStruct((_B, _MAX_LENGTH), jnp.int32),
            jax.ShapeDtypeStruct((_B, _MAX_LENGTH), jnp.int32),
        ),
        mesh=mesh,
        scratch_types=[
            pltpu.VMEM((_WALLOC,), jnp.int32),
            pltpu.VMEM((32,), jnp.int32),
            pltpu.VMEM((_HALF,), jnp.int32),
            pltpu.VMEM((_HALF,), jnp.int32),
            pltpu.SemaphoreType.DMA,
            pltpu.SemaphoreType.DMA,
        ],
    )
    input_ids, attention_mask = run(flat_tokens, cu_seqlens)
    return input_ids, attention_mask


# speedup vs baseline: 1.0573x; 1.0573x over previous
import jax
import jax.numpy as jnp
from jax import lax
from jax.experimental import pallas as pl
from jax.experimental.pallas import tpu as pltpu
from jax.experimental.pallas import tpu_sc as plsc

_MAX_LENGTH = 2048
_BOS = 49406
_EOS = 49407
_PAD = 49407
_B = 16
_TOTAL = 16384

_HALF = _MAX_LENGTH
_WIN = _HALF + 16
_WALLOC = 4128


def _body(flat_hbm, cu_hbm, ids_hbm, mask_hbm,
          win_v, cu_v, outi_v, outm_v, sem_w, sem_m):
    row = lax.axis_index("s")
    p0 = lax.axis_index("c") * _HALF

    pltpu.sync_copy(cu_hbm, cu_v.at[pl.ds(0, _B + 1)])
    cu_vec = cu_v[pl.ds(row, 16)]
    start = cu_vec[0]
    end = cu_vec[1]
    seg = jnp.minimum(end - start, _MAX_LENGTH - 2)

    aa = start + p0 - 1
    aa0 = jnp.maximum(aa, 0)
    r0 = jnp.minimum(aa0 // 16, (_TOTAL - _WIN) // 16) * 16
    off = aa - r0 + 16
    wcopy = pltpu.async_copy(
        flat_hbm.at[pl.ds(r0, _WIN)], win_v.at[pl.ds(16, _WIN)], sem_w)

    lane = lax.iota(jnp.int32, 16)
    one = jnp.int32(1)
    zero = jnp.int32(0)

    @plsc.parallel_loop(0, _HALF, step=16, unroll=8)
    def _mask_loop(i):
        p = p0 + i + lane
        outm_v[pl.ds(i, 16)] = jnp.maximum(
            jnp.minimum(seg + 2 - p, one), zero)

    mcopy = pltpu.async_copy(
        outm_v, mask_hbm.at[row, pl.ds(p0, _HALF)], sem_m)
    wcopy.wait()

    def frame(p, g):
        inb = jnp.maximum(
            jnp.minimum(jnp.minimum(p, seg + 1 - p), one), zero)
        a = jnp.minimum(p, one)
        return inb * (g - _PAD) + (a + (_PAD - 1))

    @plsc.parallel_loop(0, _HALF // 2, step=16, unroll=8)
    def _ids_loop_a(i):
        p = p0 + i + lane
        g = win_v[pl.ds(off + i, 16)]
        outi_v[pl.ds(i, 16)] = frame(p, g)

    icopy_a = pltpu.async_copy(
        outi_v.at[pl.ds(0, _HALF // 2)],
        ids_hbm.at[row, pl.ds(p0, _HALF // 2)], sem_w)

    @plsc.parallel_loop(_HALF // 2, _HALF, step=16, unroll=8)
    def _ids_loop_b(i):
        p = p0 + i + lane
        g = win_v[pl.ds(off + i, 16)]
        outi_v[pl.ds(i, 16)] = frame(p, g)

    icopy_b = pltpu.async_copy(
        outi_v.at[pl.ds(_HALF // 2, _HALF // 2)],
        ids_hbm.at[row, pl.ds(p0 + _HALF // 2, _HALF // 2)], sem_w)
    icopy_a.wait()
    icopy_b.wait()
    mcopy.wait()


@jax.jit
def kernel(flat_tokens, cu_seqlens):
    mesh = plsc.VectorSubcoreMesh(
        core_axis_name="c", subcore_axis_name="s", num_cores=1, num_subcores=16)
    run = pl.kernel(
        _body,
        out_type=(
            jax.ShapeDtypeStruct((_B, _MAX_LENGTH), jnp.int32),
            jax.ShapeDtypeStruct((_B, _MAX_LENGTH), jnp.int32),
        ),
        mesh=mesh,
        scratch_types=[
            pltpu.VMEM((_WALLOC,), jnp.int32),
            pltpu.VMEM((32,), jnp.int32),
            pltpu.VMEM((_HALF,), jnp.int32),
            pltpu.VMEM((_HALF,), jnp.int32),
            pltpu.SemaphoreType.DMA,
            pltpu.SemaphoreType.DMA,
        ],
    )
    input_ids, attention_mask = run(flat_tokens, cu_seqlens)
    return input_ids, attention_mask


# speedup vs baseline: 1.0630x; 1.0054x over previous
import jax
import jax.numpy as jnp
from jax import lax
from jax.experimental import pallas as pl
from jax.experimental.pallas import tpu as pltpu
from jax.experimental.pallas import tpu_sc as plsc

_MAX_LENGTH = 2048
_BOS = 49406
_EOS = 49407
_PAD = 49407
_B = 16
_TOTAL = 16384

_HALF = _MAX_LENGTH
_WIN = _HALF + 16
_WALLOC = 4128


def _body(flat_hbm, cu_hbm, ids_hbm, mask_hbm,
          win_v, cu_v, outi_v, outm_v, sem_w, sem_m):
    row = lax.axis_index("s")
    p0 = lax.axis_index("c") * _HALF

    pltpu.sync_copy(cu_hbm, cu_v.at[pl.ds(0, _B + 1)])
    cu_vec = cu_v[pl.ds(row, 16)]
    start = cu_vec[0]
    end = cu_vec[1]
    seg = jnp.minimum(end - start, _MAX_LENGTH - 2)

    aa = start + p0 - 1
    aa0 = jnp.maximum(aa, 0)
    r0 = jnp.minimum(aa0 // 16, (_TOTAL - _WIN) // 16) * 16
    off = aa - r0 + 16
    wcopy = pltpu.async_copy(
        flat_hbm.at[pl.ds(r0, _WIN)], win_v.at[pl.ds(16, _WIN)], sem_w)

    lane = lax.iota(jnp.int32, 16)
    one = jnp.int32(1)
    zero = jnp.int32(0)

    @plsc.parallel_loop(0, _HALF, step=16, unroll=4)
    def _mask_loop(i):
        p = p0 + i + lane
        outm_v[pl.ds(i, 16)] = jnp.maximum(
            jnp.minimum(seg + 2 - p, one), zero)

    mcopy = pltpu.async_copy(
        outm_v, mask_hbm.at[row, pl.ds(p0, _HALF)], sem_m)
    wcopy.wait()

    def frame(p, g):
        inb = jnp.maximum(
            jnp.minimum(jnp.minimum(p, seg + 1 - p), one), zero)
        a = jnp.minimum(p, one)
        return inb * (g - _PAD) + (a + (_PAD - 1))

    @plsc.parallel_loop(0, _HALF, step=16, unroll=4)
    def _ids_loop(i):
        p = p0 + i + lane
        g = win_v[pl.ds(off + i, 16)]
        outi_v[pl.ds(i, 16)] = frame(p, g)

    pltpu.sync_copy(outi_v, ids_hbm.at[row, pl.ds(p0, _HALF)])
    mcopy.wait()


@jax.jit
def kernel(flat_tokens, cu_seqlens):
    mesh = plsc.VectorSubcoreMesh(
        core_axis_name="c", subcore_axis_name="s", num_cores=1, num_subcores=16)
    run = pl.kernel(
        _body,
        out_type=(
            jax.ShapeDtypeStruct((_B, _MAX_LENGTH), jnp.int32),
            jax.ShapeDtypeStruct((_B, _MAX_LENGTH), jnp.int32),
        ),
        mesh=mesh,
        scratch_types=[
            pltpu.VMEM((_WALLOC,), jnp.int32),
            pltpu.VMEM((32,), jnp.int32),
            pltpu.VMEM((_HALF,), jnp.int32),
            pltpu.VMEM((_HALF,), jnp.int32),
            pltpu.SemaphoreType.DMA,
            pltpu.SemaphoreType.DMA,
        ],
    )
    input_ids, attention_mask = run(flat_tokens, cu_seqlens)
    return input_ids, attention_mask
